# baseline (speedup 1.0000x reference)
import functools

import jax
import jax.numpy as jnp
from jax import lax
from jax.experimental import pallas as pl
from jax.experimental.pallas import tpu as pltpu
from jax.experimental.pallas import tpu_sc as plsc

N_NODES = 4096
HID = 64
HEADS = 4
IN_DIM = 12
B = 8

NSC = 2
NTILE = 16
GCH = 64


def _silu(x):
    return x * jax.nn.sigmoid(x)


def _layernorm(x, g, b):
    m = x.mean(-1, keepdims=True)
    v = ((x - m) ** 2).mean(-1, keepdims=True)
    return (x - m) * jax.lax.rsqrt(v + 1e-5) * g + b


def _enc_body(nodes_ref, w1_ref, b1_ref, w2_ref, b2_ref, out_ref):
    n = nodes_ref[...]
    h1 = _silu(jnp.dot(n, w1_ref[...], preferred_element_type=jnp.float32)
               + b1_ref[...])
    out_ref[...] = (jnp.dot(h1, w2_ref[...], preferred_element_type=jnp.float32)
                    + b2_ref[...])


def _encode(nodes, w1, b1, w2, b2):
    rows = nodes.shape[0]
    blk = min(2048, rows)
    return pl.pallas_call(
        _enc_body,
        grid=(rows // blk,),
        in_specs=[
            pl.BlockSpec((blk, IN_DIM), lambda i: (i, 0)),
            pl.BlockSpec((IN_DIM, HID), lambda i: (0, 0)),
            pl.BlockSpec((1, HID), lambda i: (0, 0)),
            pl.BlockSpec((HID, HID), lambda i: (0, 0)),
            pl.BlockSpec((1, HID), lambda i: (0, 0)),
        ],
        out_specs=pl.BlockSpec((blk, HID), lambda i: (i, 0)),
        out_shape=jax.ShapeDtypeStruct((rows, HID), jnp.float32),
    )(nodes, w1, b1, w2, b2)


def _pre_body(za_ref, gb_ref, bias_ref, lng_ref, lnb_ref, linw_ref, ws_ref,
              wd_ref, xh_ref, as_ref, ad_ref, *, ca, cb, use_post):
    za = za_ref[...]
    if use_post:
        k = _silu(_layernorm(gb_ref[...] + bias_ref[...], lng_ref[...],
                             lnb_ref[...]))
    else:
        k = gb_ref[...]
    z = ca * za + cb * k
    xh = jnp.dot(z, linw_ref[...], preferred_element_type=jnp.float32)
    xh_ref[...] = xh
    as_ref[...] = jnp.dot(xh, ws_ref[...], preferred_element_type=jnp.float32)
    ad_ref[...] = jnp.dot(xh, wd_ref[...], preferred_element_type=jnp.float32)


def _pre(za, gb, bias, lng, lnb, linw, ws, wd, ca, cb, use_post):
    rows = za.shape[0]
    blk = min(2048, rows)
    body = functools.partial(_pre_body, ca=ca, cb=cb, use_post=use_post)
    return pl.pallas_call(
        body,
        grid=(rows // blk,),
        in_specs=[
            pl.BlockSpec((blk, HID), lambda i: (i, 0)),
            pl.BlockSpec((blk, HID), lambda i: (i, 0)),
            pl.BlockSpec((1, HID), lambda i: (0, 0)),
            pl.BlockSpec((1, HID), lambda i: (0, 0)),
            pl.BlockSpec((1, HID), lambda i: (0, 0)),
            pl.BlockSpec((HID, HEADS * HID), lambda i: (0, 0)),
            pl.BlockSpec((HEADS * HID, 16), lambda i: (0, 0)),
            pl.BlockSpec((HEADS * HID, 16), lambda i: (0, 0)),
        ],
        out_specs=[
            pl.BlockSpec((blk, HEADS * HID), lambda i: (i, 0)),
            pl.BlockSpec((blk, 16), lambda i: (i, 0)),
            pl.BlockSpec((blk, 16), lambda i: (i, 0)),
        ],
        out_shape=[
            jax.ShapeDtypeStruct((rows, HEADS * HID), jnp.float32),
            jax.ShapeDtypeStruct((rows, 16), jnp.float32),
            jax.ShapeDtypeStruct((rows, 16), jnp.float32),
        ],
    )(za, gb, bias, lng, lnb, linw, ws, wd)


def _ae_body(ed_ref, c_ref, out_ref):
    out_ref[...] = jnp.dot(ed_ref[...], c_ref[...],
                           preferred_element_type=jnp.float32)


def _ae16(ed4, cmat):
    rows = ed4.shape[0]
    blk = min(2048, rows)
    return pl.pallas_call(
        _ae_body,
        grid=(rows // blk,),
        in_specs=[
            pl.BlockSpec((blk, 4), lambda i: (i, 0)),
            pl.BlockSpec((4, 16), lambda i: (0, 0)),
        ],
        out_specs=pl.BlockSpec((blk, 16), lambda i: (i, 0)),
        out_shape=jax.ShapeDtypeStruct((rows, 16), jnp.float32),
    )(ed4, cmat)


def _sc_edge_body(xh_hbm, asds_hbm, asdd_hbm, ae_hbm, src_hbm, dst_hbm,
                  out_hbm, srcv, dstv, srcg2, dst2v,
                  aev, asv, adv, xrow, xrow2, msgbuf, msgbuf2, z16, z64,
                  asds_sh, asdd_sh, den_sh, out_sh, sem, sem2):
    c = lax.axis_index("c")
    s = lax.axis_index("s")
    ept = srcv.shape[0]
    nch = ept // GCH
    nrows = N_NODES // NTILE
    base_e = s * ept

    pltpu.sync_copy(src_hbm.at[pl.ds(base_e, ept)], srcv)
    pltpu.sync_copy(dst_hbm.at[pl.ds(base_e, ept)], dstv)
    pltpu.sync_copy(ae_hbm.at[pl.ds(base_e, ept)], aev)

    def d2loop(k, carry):
        for j in range(GCH // 16):
            dst2v[k, pl.ds(j * 16, 16)] = dstv[pl.ds(k * GCH + j * 16, 16)]
        return carry

    lax.fori_loop(0, ept // GCH, d2loop, 0)

    def zloop16(i, carry):
        z16[i] = jnp.zeros((16,), jnp.float32)
        return carry

    lax.fori_loop(0, z16.shape[0], zloop16, 0)

    def zloop64(i, carry):
        for j in range(HID // 16):
            z64[i, pl.ds(j * 16, 16)] = jnp.zeros((16,), jnp.float32)
        return carry

    lax.fori_loop(0, z64.shape[0], zloop64, 0)

    def batch_body(b, carry):
        bg = c * (B // NSC) + b
        off = bg * N_NODES

        def idx2loop(k, cc):
            for j in range(GCH // 16):
                srcg2[k, pl.ds(j * 16, 16)] = (
                    srcv[pl.ds(k * GCH + j * 16, 16)] + off)
            return cc

        lax.fori_loop(0, nch, idx2loop, 0)

        cps = [
            pltpu.async_copy(z16, den_sh.at[pl.ds(s * nrows, nrows)], sem),
        ]
        for j in range(nrows // z64.shape[0]):
            cps.append(pltpu.async_copy(
                z64, out_sh.at[pl.ds(s * nrows + j * z64.shape[0],
                                     z64.shape[0])], sem))
        cps.append(pltpu.async_copy(
            asds_hbm.at[pl.ds(off + s * nrows, nrows)],
            asds_sh.at[pl.ds(s * nrows, nrows)], sem))
        cps.append(pltpu.async_copy(
            asdd_hbm.at[pl.ds(off + s * nrows, nrows)],
            asdd_sh.at[pl.ds(s * nrows, nrows)], sem))
        for cp in cps:
            cp.wait()
        plsc.subcore_barrier()

        cp_s = pltpu.async_copy(asds_sh.at[srcv], asv, sem)
        cp_d = pltpu.async_copy(asdd_sh.at[dstv], adv, sem2)
        cp_s.wait()
        cp_d.wait()

        def exloop(e8, cc):
            for u in range(8):
                e = e8 * 8 + u
                a = asv[e] + adv[e] + aev[e]
                a = jnp.where(a >= 0, a, 0.2 * a)
                asv[e] = jnp.exp(a)
            return cc

        lax.fori_loop(0, ept // 8, exloop, 0)

        pltpu.sync_copy(asv, den_sh.at[dstv], add=True)
        plsc.subcore_barrier()

        pltpu.async_copy(den_sh.at[dstv], adv, sem).wait()

        def compute_chunk(k, buf, mbuf):
            for i in range(GCH):
                e = k * GCH + i
                wrow = asv[e] / (4.0 * adv[e] + 4e-16)
                wh = [jnp.broadcast_to(lax.slice(wrow, (h,), (h + 1,)), (16,))
                      for h in range(HEADS)]
                for j in range(HID // 16):
                    acc = wh[0] * buf[i, pl.ds(j * 16, 16)]
                    for h in range(1, HEADS):
                        acc = acc + wh[h] * buf[i, pl.ds(h * HID + j * 16, 16)]
                    mbuf[i, pl.ds(j * 16, 16)] = acc
            pltpu.sync_copy(mbuf, out_sh.at[dst2v.at[k]], add=True)

        pltpu.async_copy(xh_hbm.at[srcg2.at[0]], xrow, sem)

        def pair_body(p, cc):
            ka = 2 * p
            kb = 2 * p + 1
            pltpu.make_async_copy(xh_hbm.at[srcg2.at[0]], xrow, sem).wait()
            pltpu.async_copy(xh_hbm.at[srcg2.at[kb]], xrow2, sem2)
            compute_chunk(ka, xrow, msgbuf)
            pltpu.make_async_copy(xh_hbm.at[srcg2.at[0]], xrow2, sem2).wait()
            knext = jnp.minimum(ka + 2, nch - 1)
            pltpu.async_copy(xh_hbm.at[srcg2.at[knext]], xrow, sem)
            compute_chunk(kb, xrow2, msgbuf2)
            return cc

        lax.fori_loop(0, nch // 2, pair_body, 0)
        pltpu.make_async_copy(xh_hbm.at[srcg2.at[0]], xrow, sem).wait()
        plsc.subcore_barrier()

        pltpu.sync_copy(out_sh.at[pl.ds(s * nrows, nrows)],
                        out_hbm.at[pl.ds(off + s * nrows, nrows)])
        return carry

    lax.fori_loop(0, B // NSC, batch_body, 0)


def _sc_edge(xh, asds, asdd, ae16, src_i, dst_i):
    bn = xh.shape[0]
    E = src_i.shape[0]
    ept = E // NTILE
    fn = pl.kernel(
        _sc_edge_body,
        out_type=pltpu.MemorySpace.HBM((bn, HID), jnp.float32),
        mesh=plsc.VectorSubcoreMesh(core_axis_name="c", subcore_axis_name="s"),
        compiler_params=pltpu.CompilerParams(use_tc_tiling_on_sc=False),
        scratch_types=[
            pltpu.VMEM((ept,), jnp.int32),
            pltpu.VMEM((ept,), jnp.int32),
            pltpu.VMEM((ept // GCH, GCH), jnp.int32),
            pltpu.VMEM((ept // GCH, GCH), jnp.int32),
            pltpu.VMEM((ept, 16), jnp.float32),
            pltpu.VMEM((ept, 16), jnp.float32),
            pltpu.VMEM((ept, 16), jnp.float32),
            pltpu.VMEM((GCH, HEADS * HID), jnp.float32),
            pltpu.VMEM((GCH, HEADS * HID), jnp.float32),
            pltpu.VMEM((GCH, HID), jnp.float32),
            pltpu.VMEM((GCH, HID), jnp.float32),
            pltpu.VMEM((N_NODES // NTILE, 16), jnp.float32),
            pltpu.VMEM((GCH, HID), jnp.float32),
            pltpu.VMEM_SHARED((N_NODES, 16), jnp.float32),
            pltpu.VMEM_SHARED((N_NODES, 16), jnp.float32),
            pltpu.VMEM_SHARED((N_NODES, 16), jnp.float32),
            pltpu.VMEM_SHARED((N_NODES, HID), jnp.float32),
            pltpu.SemaphoreType.DMA,
            pltpu.SemaphoreType.DMA,
        ],
    )
    return fn(xh, asds, asdd, ae16, src_i, dst_i)


def _fin_body(h_ref, g1_ref, g2_ref, g3_ref, g4_ref, bias_ref, lng_ref,
              lnb_ref, out_ref):
    def post(g):
        return _silu(_layernorm(g + bias_ref[...], lng_ref[...], lnb_ref[...]))

    k1 = post(g1_ref[...])
    k2 = post(g2_ref[...])
    k3 = post(g3_ref[...])
    k4 = post(g4_ref[...])
    out_ref[...] = h_ref[...] + (1.0 / 6.0) * (k1 + 2 * k2 + 2 * k3 + k4)


def _fin(h, g1, g2, g3, g4, bias, lng, lnb):
    rows = h.shape[0]
    blk = min(2048, rows)
    row_spec = pl.BlockSpec((blk, HID), lambda i: (i, 0))
    vec_spec = pl.BlockSpec((1, HID), lambda i: (0, 0))
    return pl.pallas_call(
        _fin_body,
        grid=(rows // blk,),
        in_specs=[row_spec] * 5 + [vec_spec] * 3,
        out_specs=row_spec,
        out_shape=jax.ShapeDtypeStruct((rows, HID), jnp.float32),
    )(h, g1, g2, g3, g4, bias, lng, lnb)


def _head_body(h_ref, flag_ref, lng_ref, lnb_ref, w1_ref, b1_ref, w2_ref,
               b2_ref, out_ref):
    z = _layernorm(h_ref[...], lng_ref[...], lnb_ref[...])
    z = _silu(jnp.dot(z, w1_ref[...], preferred_element_type=jnp.float32)
              + b1_ref[...])
    logits = (z * w2_ref[...]).sum(-1, keepdims=True) + b2_ref[...]
    out_ref[...] = jnp.where(flag_ref[...] > 0.5,
                             jnp.maximum(logits, 6.0), logits)


def _head(h, flag, lng, lnb, w1, b1, w2, b2):
    rows = h.shape[0]
    blk = min(2048, rows)
    return pl.pallas_call(
        _head_body,
        grid=(rows // blk,),
        in_specs=[
            pl.BlockSpec((blk, HID), lambda i: (i, 0)),
            pl.BlockSpec((blk, 1), lambda i: (i, 0)),
            pl.BlockSpec((1, HID), lambda i: (0, 0)),
            pl.BlockSpec((1, HID), lambda i: (0, 0)),
            pl.BlockSpec((HID, HID), lambda i: (0, 0)),
            pl.BlockSpec((1, HID), lambda i: (0, 0)),
            pl.BlockSpec((1, HID), lambda i: (0, 0)),
            pl.BlockSpec((1, 1), lambda i: (0, 0)),
        ],
        out_specs=pl.BlockSpec((blk, 1), lambda i: (i, 0)),
        out_shape=jax.ShapeDtypeStruct((rows, 1), jnp.float32),
    )(h, flag, lng, lnb, w1, b1, w2, b2)


def kernel(x, edge_index, edge_dirs, params):
    E = edge_index.shape[1]
    n_rows = B * N_NODES
    nodes = x.reshape(B, IN_DIM, N_NODES).transpose(0, 2, 1)
    nodes = nodes.reshape(n_rows, IN_DIM).astype(jnp.float32)

    src_i = edge_index[0].astype(jnp.int32)
    dst_i = edge_index[1].astype(jnp.int32)
    ed4 = jnp.concatenate(
        [edge_dirs, jnp.zeros((E, 1), jnp.float32)], axis=1)

    p = params
    gat_pack = []
    for gp in p['gats']:
        ws = jnp.zeros((HEADS * HID, 16), jnp.float32)
        wd = jnp.zeros((HEADS * HID, 16), jnp.float32)
        for h in range(HEADS):
            ws = ws.at[h * HID:(h + 1) * HID, h].set(gp['att_src'][h])
            wd = wd.at[h * HID:(h + 1) * HID, h].set(gp['att_dst'][h])
        lew = gp['lin_edge_W'].reshape(3, HEADS, HID)
        c34 = jnp.einsum('jhk,hk->jh', lew, gp['att_edge'])
        cmat = jnp.zeros((4, 16), jnp.float32).at[:3, :4].set(c34)
        gat_pack.append({
            'lin_W': gp['lin_W'], 'ws': ws, 'wd': wd, 'cmat': cmat,
            'bias': gp['bias'].reshape(1, HID),
            'ln_g': gp['ln_g'].reshape(1, HID),
            'ln_b': gp['ln_b'].reshape(1, HID),
        })

    ae_l = [_ae16(ed4, gat_pack[0]['cmat']), _ae16(ed4, gat_pack[1]['cmat'])]

    h0 = _encode(nodes, p['enc_W1'], p['enc_b1'].reshape(1, HID),
                 p['enc_W2'], p['enc_b2'].reshape(1, HID))

    zeros_g = jnp.zeros((n_rows, HID), jnp.float32)
    coefs = [(1.0, 0.0), (1.0, 0.5), (1.0, 0.5), (1.0, 1.0)]
    g2_list = []
    g2_prev = zeros_g
    for i in range(4):
        ca, cb = coefs[i]
        gp0, gp1 = gat_pack
        xh, asds, asdd = _pre(h0, g2_prev, gp1['bias'], gp1['ln_g'],
                              gp1['ln_b'], gp0['lin_W'], gp0['ws'], gp0['wd'],
                              ca, cb, use_post=(i != 0))
        g1 = _sc_edge(xh, asds, asdd, ae_l[0], src_i, dst_i)
        xh2, asds2, asdd2 = _pre(zeros_g, g1, gp0['bias'], gp0['ln_g'],
                                 gp0['ln_b'], gp1['lin_W'], gp1['ws'],
                                 gp1['wd'], 0.0, 1.0, use_post=True)
        g2 = _sc_edge(xh2, asds2, asdd2, ae_l[1], src_i, dst_i)
        g2_list.append(g2)
        g2_prev = g2

    gp1 = gat_pack[1]
    h_fin = _fin(h0, g2_list[0], g2_list[1], g2_list[2], g2_list[3],
                 gp1['bias'], gp1['ln_g'], gp1['ln_b'])

    flag = x[:, 0].reshape(n_rows, 1)
    logits = _head(h_fin, flag, p['head_ln_g'].reshape(1, HID),
                   p['head_ln_b'].reshape(1, HID), p['head_W1'],
                   p['head_b1'].reshape(1, HID),
                   p['head_W2'].reshape(1, HID),
                   p['head_b2'].reshape(1, 1))
    return logits.reshape(B, x.shape[2], x.shape[3])


# speedup vs baseline: 1.2082x; 1.2082x over previous
import functools

import jax
import jax.numpy as jnp
from jax import lax
from jax.experimental import pallas as pl
from jax.experimental.pallas import tpu as pltpu
from jax.experimental.pallas import tpu_sc as plsc

N_NODES = 4096
HID = 64
HEADS = 4
IN_DIM = 12
B = 8

NSC = 2
NTILE = 16
GCH = 64


def _silu(x):
    return x * jax.nn.sigmoid(x)


def _layernorm(x, g, b):
    m = x.mean(-1, keepdims=True)
    v = ((x - m) ** 2).mean(-1, keepdims=True)
    return (x - m) * jax.lax.rsqrt(v + 1e-5) * g + b


def _enc_body(nodes_ref, w1_ref, b1_ref, w2_ref, b2_ref, out_ref):
    n = nodes_ref[...]
    h1 = _silu(jnp.dot(n, w1_ref[...], preferred_element_type=jnp.float32)
               + b1_ref[...])
    out_ref[...] = (jnp.dot(h1, w2_ref[...], preferred_element_type=jnp.float32)
                    + b2_ref[...])


def _encode(nodes, w1, b1, w2, b2):
    rows = nodes.shape[0]
    blk = min(2048, rows)
    return pl.pallas_call(
        _enc_body,
        grid=(rows // blk,),
        in_specs=[
            pl.BlockSpec((blk, IN_DIM), lambda i: (i, 0)),
            pl.BlockSpec((IN_DIM, HID), lambda i: (0, 0)),
            pl.BlockSpec((1, HID), lambda i: (0, 0)),
            pl.BlockSpec((HID, HID), lambda i: (0, 0)),
            pl.BlockSpec((1, HID), lambda i: (0, 0)),
        ],
        out_specs=pl.BlockSpec((blk, HID), lambda i: (i, 0)),
        out_shape=jax.ShapeDtypeStruct((rows, HID), jnp.float32),
    )(nodes, w1, b1, w2, b2)


def _pre_body(za_ref, gb_ref, bias_ref, lng_ref, lnb_ref, linw_ref, ws_ref,
              wd_ref, xh_ref, as_ref, ad_ref, *, ca, cb, use_post):
    za = za_ref[...]
    if use_post:
        k = _silu(_layernorm(gb_ref[...] + bias_ref[...], lng_ref[...],
                             lnb_ref[...]))
    else:
        k = gb_ref[...]
    z = ca * za + cb * k
    xh = jnp.dot(z, linw_ref[...], preferred_element_type=jnp.float32)
    xh_ref[...] = xh
    as_ref[...] = jnp.dot(xh, ws_ref[...], preferred_element_type=jnp.float32)
    ad_ref[...] = jnp.dot(xh, wd_ref[...], preferred_element_type=jnp.float32)


def _pre(za, gb, bias, lng, lnb, linw, ws, wd, ca, cb, use_post):
    rows = za.shape[0]
    blk = min(2048, rows)
    body = functools.partial(_pre_body, ca=ca, cb=cb, use_post=use_post)
    return pl.pallas_call(
        body,
        grid=(rows // blk,),
        in_specs=[
            pl.BlockSpec((blk, HID), lambda i: (i, 0)),
            pl.BlockSpec((blk, HID), lambda i: (i, 0)),
            pl.BlockSpec((1, HID), lambda i: (0, 0)),
            pl.BlockSpec((1, HID), lambda i: (0, 0)),
            pl.BlockSpec((1, HID), lambda i: (0, 0)),
            pl.BlockSpec((HID, HEADS * HID), lambda i: (0, 0)),
            pl.BlockSpec((HEADS * HID, 16), lambda i: (0, 0)),
            pl.BlockSpec((HEADS * HID, 16), lambda i: (0, 0)),
        ],
        out_specs=[
            pl.BlockSpec((blk, HEADS * HID), lambda i: (i, 0)),
            pl.BlockSpec((blk, 16), lambda i: (i, 0)),
            pl.BlockSpec((blk, 16), lambda i: (i, 0)),
        ],
        out_shape=[
            jax.ShapeDtypeStruct((rows, HEADS * HID), jnp.float32),
            jax.ShapeDtypeStruct((rows, 16), jnp.float32),
            jax.ShapeDtypeStruct((rows, 16), jnp.float32),
        ],
    )(za, gb, bias, lng, lnb, linw, ws, wd)


def _ae_body(ed_ref, c_ref, out_ref):
    out_ref[...] = jnp.dot(ed_ref[...], c_ref[...],
                           preferred_element_type=jnp.float32)


def _ae16(ed4, cmat):
    rows = ed4.shape[0]
    blk = min(2048, rows)
    return pl.pallas_call(
        _ae_body,
        grid=(rows // blk,),
        in_specs=[
            pl.BlockSpec((blk, 4), lambda i: (i, 0)),
            pl.BlockSpec((4, 16), lambda i: (0, 0)),
        ],
        out_specs=pl.BlockSpec((blk, 16), lambda i: (i, 0)),
        out_shape=jax.ShapeDtypeStruct((rows, 16), jnp.float32),
    )(ed4, cmat)


def _sc_edge_body(xh_hbm, asds_hbm, asdd_hbm, ae_hbm, src_hbm, dst_hbm,
                  out_hbm, srcv, dstv, srcg2, dst2v,
                  aev, asv, adv, xrow, xrow2, msgbuf, msgbuf2, z16, z64,
                  asds_sh, asdd_sh, den_sh, out_sh, sem, sem2):
    c = lax.axis_index("c")
    s = lax.axis_index("s")
    ept = srcv.shape[0]
    nch = ept // GCH
    nrows = N_NODES // NTILE
    base_e = s * ept

    pltpu.sync_copy(src_hbm.at[pl.ds(base_e, ept)], srcv)
    pltpu.sync_copy(dst_hbm.at[pl.ds(base_e, ept)], dstv)
    pltpu.sync_copy(ae_hbm.at[pl.ds(base_e, ept)], aev)

    def d2loop(k, carry):
        for j in range(GCH // 16):
            dst2v[k, pl.ds(j * 16, 16)] = dstv[pl.ds(k * GCH + j * 16, 16)]
        return carry

    lax.fori_loop(0, ept // GCH, d2loop, 0)

    def zloop16(i, carry):
        z16[i] = jnp.zeros((16,), jnp.float32)
        return carry

    lax.fori_loop(0, z16.shape[0], zloop16, 0)

    def zloop64(i, carry):
        for j in range(HID // 16):
            z64[i, pl.ds(j * 16, 16)] = jnp.zeros((16,), jnp.float32)
        return carry

    lax.fori_loop(0, z64.shape[0], zloop64, 0)

    def batch_body(b, carry):
        bg = c * (B // NSC) + b
        off = bg * N_NODES

        def idx2loop(k, cc):
            for j in range(GCH // 16):
                srcg2[k, pl.ds(j * 16, 16)] = (
                    srcv[pl.ds(k * GCH + j * 16, 16)] + off)
            return cc

        lax.fori_loop(0, nch, idx2loop, 0)

        cps = [
            pltpu.async_copy(z16, den_sh.at[pl.ds(s * nrows, nrows)], sem),
        ]
        for j in range(nrows // z64.shape[0]):
            cps.append(pltpu.async_copy(
                z64, out_sh.at[pl.ds(s * nrows + j * z64.shape[0],
                                     z64.shape[0])], sem))
        cps.append(pltpu.async_copy(
            asds_hbm.at[pl.ds(off + s * nrows, nrows)],
            asds_sh.at[pl.ds(s * nrows, nrows)], sem))
        cps.append(pltpu.async_copy(
            asdd_hbm.at[pl.ds(off + s * nrows, nrows)],
            asdd_sh.at[pl.ds(s * nrows, nrows)], sem))
        for cp in cps:
            cp.wait()
        plsc.subcore_barrier()

        cp_s = pltpu.async_copy(asds_sh.at[srcv], asv, sem)
        cp_d = pltpu.async_copy(asdd_sh.at[dstv], adv, sem2)
        cp_s.wait()
        cp_d.wait()

        def exloop(e8, cc):
            for u in range(8):
                e = e8 * 8 + u
                a = asv[e] + adv[e] + aev[e]
                a = jnp.where(a >= 0, a, 0.2 * a)
                asv[e] = jnp.exp(a)
            return cc

        lax.fori_loop(0, ept // 8, exloop, 0)

        pltpu.sync_copy(asv, den_sh.at[dstv], add=True)
        plsc.subcore_barrier()

        pltpu.async_copy(den_sh.at[dstv], adv, sem).wait()

        def wloop(e8, cc):
            for u in range(8):
                e = e8 * 8 + u
                asv[e] = asv[e] / (4.0 * adv[e] + 4e-16)
            return cc

        lax.fori_loop(0, ept // 8, wloop, 0)

        def compute_chunk(k, buf, mbuf):
            for i in range(GCH):
                wrow = asv[k * GCH + i]
                wh = [jnp.broadcast_to(lax.slice(wrow, (h,), (h + 1,)), (16,))
                      for h in range(HEADS)]
                for j in range(HID // 16):
                    acc = wh[0] * buf[i, pl.ds(j * 16, 16)]
                    for h in range(1, HEADS):
                        acc = acc + wh[h] * buf[i, pl.ds(h * HID + j * 16, 16)]
                    mbuf[i, pl.ds(j * 16, 16)] = acc
            pltpu.sync_copy(mbuf, out_sh.at[dst2v.at[k]], add=True)

        pltpu.async_copy(xh_hbm.at[srcg2.at[0]], xrow, sem)

        def pair_body(p, cc):
            ka = 2 * p
            kb = 2 * p + 1
            pltpu.make_async_copy(xh_hbm.at[srcg2.at[0]], xrow, sem).wait()
            pltpu.async_copy(xh_hbm.at[srcg2.at[kb]], xrow2, sem2)
            compute_chunk(ka, xrow, msgbuf)
            pltpu.make_async_copy(xh_hbm.at[srcg2.at[0]], xrow2, sem2).wait()
            knext = jnp.minimum(ka + 2, nch - 1)
            pltpu.async_copy(xh_hbm.at[srcg2.at[knext]], xrow, sem)
            compute_chunk(kb, xrow2, msgbuf2)
            return cc

        lax.fori_loop(0, nch // 2, pair_body, 0)
        pltpu.make_async_copy(xh_hbm.at[srcg2.at[0]], xrow, sem).wait()
        plsc.subcore_barrier()

        pltpu.sync_copy(out_sh.at[pl.ds(s * nrows, nrows)],
                        out_hbm.at[pl.ds(off + s * nrows, nrows)])
        return carry

    lax.fori_loop(0, B // NSC, batch_body, 0)


def _sc_edge(xh, asds, asdd, ae16, src_i, dst_i):
    bn = xh.shape[0]
    E = src_i.shape[0]
    ept = E // NTILE
    fn = pl.kernel(
        _sc_edge_body,
        out_type=pltpu.MemorySpace.HBM((bn, HID), jnp.float32),
        mesh=plsc.VectorSubcoreMesh(core_axis_name="c", subcore_axis_name="s"),
        compiler_params=pltpu.CompilerParams(use_tc_tiling_on_sc=False),
        scratch_types=[
            pltpu.VMEM((ept,), jnp.int32),
            pltpu.VMEM((ept,), jnp.int32),
            pltpu.VMEM((ept // GCH, GCH), jnp.int32),
            pltpu.VMEM((ept // GCH, GCH), jnp.int32),
            pltpu.VMEM((ept, 16), jnp.float32),
            pltpu.VMEM((ept, 16), jnp.float32),
            pltpu.VMEM((ept, 16), jnp.float32),
            pltpu.VMEM((GCH, HEADS * HID), jnp.float32),
            pltpu.VMEM((GCH, HEADS * HID), jnp.float32),
            pltpu.VMEM((GCH, HID), jnp.float32),
            pltpu.VMEM((GCH, HID), jnp.float32),
            pltpu.VMEM((N_NODES // NTILE, 16), jnp.float32),
            pltpu.VMEM((GCH, HID), jnp.float32),
            pltpu.VMEM_SHARED((N_NODES, 16), jnp.float32),
            pltpu.VMEM_SHARED((N_NODES, 16), jnp.float32),
            pltpu.VMEM_SHARED((N_NODES, 16), jnp.float32),
            pltpu.VMEM_SHARED((N_NODES, HID), jnp.float32),
            pltpu.SemaphoreType.DMA,
            pltpu.SemaphoreType.DMA,
        ],
    )
    return fn(xh, asds, asdd, ae16, src_i, dst_i)


def _fin_body(h_ref, g1_ref, g2_ref, g3_ref, g4_ref, bias_ref, lng_ref,
              lnb_ref, out_ref):
    def post(g):
        return _silu(_layernorm(g + bias_ref[...], lng_ref[...], lnb_ref[...]))

    k1 = post(g1_ref[...])
    k2 = post(g2_ref[...])
    k3 = post(g3_ref[...])
    k4 = post(g4_ref[...])
    out_ref[...] = h_ref[...] + (1.0 / 6.0) * (k1 + 2 * k2 + 2 * k3 + k4)


def _fin(h, g1, g2, g3, g4, bias, lng, lnb):
    rows = h.shape[0]
    blk = min(2048, rows)
    row_spec = pl.BlockSpec((blk, HID), lambda i: (i, 0))
    vec_spec = pl.BlockSpec((1, HID), lambda i: (0, 0))
    return pl.pallas_call(
        _fin_body,
        grid=(rows // blk,),
        in_specs=[row_spec] * 5 + [vec_spec] * 3,
        out_specs=row_spec,
        out_shape=jax.ShapeDtypeStruct((rows, HID), jnp.float32),
    )(h, g1, g2, g3, g4, bias, lng, lnb)


def _head_body(h_ref, flag_ref, lng_ref, lnb_ref, w1_ref, b1_ref, w2_ref,
               b2_ref, out_ref):
    z = _layernorm(h_ref[...], lng_ref[...], lnb_ref[...])
    z = _silu(jnp.dot(z, w1_ref[...], preferred_element_type=jnp.float32)
              + b1_ref[...])
    logits = (z * w2_ref[...]).sum(-1, keepdims=True) + b2_ref[...]
    out_ref[...] = jnp.where(flag_ref[...] > 0.5,
                             jnp.maximum(logits, 6.0), logits)


def _head(h, flag, lng, lnb, w1, b1, w2, b2):
    rows = h.shape[0]
    blk = min(2048, rows)
    return pl.pallas_call(
        _head_body,
        grid=(rows // blk,),
        in_specs=[
            pl.BlockSpec((blk, HID), lambda i: (i, 0)),
            pl.BlockSpec((blk, 1), lambda i: (i, 0)),
            pl.BlockSpec((1, HID), lambda i: (0, 0)),
            pl.BlockSpec((1, HID), lambda i: (0, 0)),
            pl.BlockSpec((HID, HID), lambda i: (0, 0)),
            pl.BlockSpec((1, HID), lambda i: (0, 0)),
            pl.BlockSpec((1, HID), lambda i: (0, 0)),
            pl.BlockSpec((1, 1), lambda i: (0, 0)),
        ],
        out_specs=pl.BlockSpec((blk, 1), lambda i: (i, 0)),
        out_shape=jax.ShapeDtypeStruct((rows, 1), jnp.float32),
    )(h, flag, lng, lnb, w1, b1, w2, b2)


def kernel(x, edge_index, edge_dirs, params):
    E = edge_index.shape[1]
    n_rows = B * N_NODES
    nodes = x.reshape(B, IN_DIM, N_NODES).transpose(0, 2, 1)
    nodes = nodes.reshape(n_rows, IN_DIM).astype(jnp.float32)

    src_i = edge_index[0].astype(jnp.int32)
    dst_i = edge_index[1].astype(jnp.int32)
    ed4 = jnp.concatenate(
        [edge_dirs, jnp.zeros((E, 1), jnp.float32)], axis=1)

    p = params
    gat_pack = []
    for gp in p['gats']:
        ws = jnp.zeros((HEADS * HID, 16), jnp.float32)
        wd = jnp.zeros((HEADS * HID, 16), jnp.float32)
        for h in range(HEADS):
            ws = ws.at[h * HID:(h + 1) * HID, h].set(gp['att_src'][h])
            wd = wd.at[h * HID:(h + 1) * HID, h].set(gp['att_dst'][h])
        lew = gp['lin_edge_W'].reshape(3, HEADS, HID)
        c34 = jnp.einsum('jhk,hk->jh', lew, gp['att_edge'])
        cmat = jnp.zeros((4, 16), jnp.float32).at[:3, :4].set(c34)
        gat_pack.append({
            'lin_W': gp['lin_W'], 'ws': ws, 'wd': wd, 'cmat': cmat,
            'bias': gp['bias'].reshape(1, HID),
            'ln_g': gp['ln_g'].reshape(1, HID),
            'ln_b': gp['ln_b'].reshape(1, HID),
        })

    ae_l = [_ae16(ed4, gat_pack[0]['cmat']), _ae16(ed4, gat_pack[1]['cmat'])]

    h0 = _encode(nodes, p['enc_W1'], p['enc_b1'].reshape(1, HID),
                 p['enc_W2'], p['enc_b2'].reshape(1, HID))

    zeros_g = jnp.zeros((n_rows, HID), jnp.float32)
    coefs = [(1.0, 0.0), (1.0, 0.5), (1.0, 0.5), (1.0, 1.0)]
    g2_list = []
    g2_prev = zeros_g
    for i in range(4):
        ca, cb = coefs[i]
        gp0, gp1 = gat_pack
        xh, asds, asdd = _pre(h0, g2_prev, gp1['bias'], gp1['ln_g'],
                              gp1['ln_b'], gp0['lin_W'], gp0['ws'], gp0['wd'],
                              ca, cb, use_post=(i != 0))
        g1 = _sc_edge(xh, asds, asdd, ae_l[0], src_i, dst_i)
        xh2, asds2, asdd2 = _pre(zeros_g, g1, gp0['bias'], gp0['ln_g'],
                                 gp0['ln_b'], gp1['lin_W'], gp1['ws'],
                                 gp1['wd'], 0.0, 1.0, use_post=True)
        g2 = _sc_edge(xh2, asds2, asdd2, ae_l[1], src_i, dst_i)
        g2_list.append(g2)
        g2_prev = g2

    gp1 = gat_pack[1]
    h_fin = _fin(h0, g2_list[0], g2_list[1], g2_list[2], g2_list[3],
                 gp1['bias'], gp1['ln_g'], gp1['ln_b'])

    flag = x[:, 0].reshape(n_rows, 1)
    logits = _head(h_fin, flag, p['head_ln_g'].reshape(1, HID),
                   p['head_ln_b'].reshape(1, HID), p['head_W1'],
                   p['head_b1'].reshape(1, HID),
                   p['head_W2'].reshape(1, HID),
                   p['head_b2'].reshape(1, 1))
    return logits.reshape(B, x.shape[2], x.shape[3])


# speedup vs baseline: 1.2335x; 1.0210x over previous
import functools

import jax
import jax.numpy as jnp
from jax import lax
from jax.experimental import pallas as pl
from jax.experimental.pallas import tpu as pltpu
from jax.experimental.pallas import tpu_sc as plsc

N_NODES = 4096
HID = 64
HEADS = 4
IN_DIM = 12
B = 8

NSC = 2
NTILE = 16
GCH = 64


def _silu(x):
    return x * jax.nn.sigmoid(x)


def _layernorm(x, g, b):
    m = x.mean(-1, keepdims=True)
    v = ((x - m) ** 2).mean(-1, keepdims=True)
    return (x - m) * jax.lax.rsqrt(v + 1e-5) * g + b


def _enc_body(nodes_ref, w1_ref, b1_ref, w2_ref, b2_ref, out_ref):
    n = nodes_ref[...]
    h1 = _silu(jnp.dot(n, w1_ref[...], preferred_element_type=jnp.float32)
               + b1_ref[...])
    out_ref[...] = (jnp.dot(h1, w2_ref[...], preferred_element_type=jnp.float32)
                    + b2_ref[...])


def _encode(nodes, w1, b1, w2, b2):
    rows = nodes.shape[0]
    blk = min(2048, rows)
    return pl.pallas_call(
        _enc_body,
        grid=(rows // blk,),
        in_specs=[
            pl.BlockSpec((blk, IN_DIM), lambda i: (i, 0)),
            pl.BlockSpec((IN_DIM, HID), lambda i: (0, 0)),
            pl.BlockSpec((1, HID), lambda i: (0, 0)),
            pl.BlockSpec((HID, HID), lambda i: (0, 0)),
            pl.BlockSpec((1, HID), lambda i: (0, 0)),
        ],
        out_specs=pl.BlockSpec((blk, HID), lambda i: (i, 0)),
        out_shape=jax.ShapeDtypeStruct((rows, HID), jnp.float32),
    )(nodes, w1, b1, w2, b2)


def _pre_body(*refs, ca, cb, mode):
    if mode == "plain":
        (za_ref, linw_ref, ws_ref, wd_ref,
         xh_ref, as_ref, ad_ref) = refs
        z = za_ref[...]
    elif mode == "post":
        (gb_ref, bias_ref, lng_ref, lnb_ref, linw_ref, ws_ref, wd_ref,
         xh_ref, as_ref, ad_ref) = refs
        z = cb * _silu(_layernorm(gb_ref[...] + bias_ref[...], lng_ref[...],
                                  lnb_ref[...]))
    else:
        (za_ref, gb_ref, bias_ref, lng_ref, lnb_ref, linw_ref, ws_ref,
         wd_ref, xh_ref, as_ref, ad_ref) = refs
        k = _silu(_layernorm(gb_ref[...] + bias_ref[...], lng_ref[...],
                             lnb_ref[...]))
        z = ca * za_ref[...] + cb * k
    xh = jnp.dot(z, linw_ref[...], preferred_element_type=jnp.float32)
    xh_ref[...] = xh
    as_ref[...] = jnp.dot(xh, ws_ref[...], preferred_element_type=jnp.float32)
    ad_ref[...] = jnp.dot(xh, wd_ref[...], preferred_element_type=jnp.float32)


def _pre(za, gb, bias, lng, lnb, linw, ws, wd, ca, cb, mode):
    rows = (za if za is not None else gb).shape[0]
    blk = min(2048, rows)
    body = functools.partial(_pre_body, ca=ca, cb=cb, mode=mode)
    row_spec = pl.BlockSpec((blk, HID), lambda i: (i, 0))
    vec_spec = pl.BlockSpec((1, HID), lambda i: (0, 0))
    w_specs = [
        pl.BlockSpec((HID, HEADS * HID), lambda i: (0, 0)),
        pl.BlockSpec((HEADS * HID, 16), lambda i: (0, 0)),
        pl.BlockSpec((HEADS * HID, 16), lambda i: (0, 0)),
    ]
    if mode == "plain":
        in_specs = [row_spec] + w_specs
        args = (za, linw, ws, wd)
    elif mode == "post":
        in_specs = [row_spec] + [vec_spec] * 3 + w_specs
        args = (gb, bias, lng, lnb, linw, ws, wd)
    else:
        in_specs = [row_spec, row_spec] + [vec_spec] * 3 + w_specs
        args = (za, gb, bias, lng, lnb, linw, ws, wd)
    return pl.pallas_call(
        body,
        grid=(rows // blk,),
        in_specs=in_specs,
        out_specs=[
            pl.BlockSpec((blk, HEADS * HID), lambda i: (i, 0)),
            pl.BlockSpec((blk, 16), lambda i: (i, 0)),
            pl.BlockSpec((blk, 16), lambda i: (i, 0)),
        ],
        out_shape=[
            jax.ShapeDtypeStruct((rows, HEADS * HID), jnp.float32),
            jax.ShapeDtypeStruct((rows, 16), jnp.float32),
            jax.ShapeDtypeStruct((rows, 16), jnp.float32),
        ],
    )(*args)


def _ae_body(ed_ref, c_ref, out_ref):
    out_ref[...] = jnp.dot(ed_ref[...], c_ref[...],
                           preferred_element_type=jnp.float32)


def _ae16(ed4, cmat):
    rows = ed4.shape[0]
    blk = min(2048, rows)
    return pl.pallas_call(
        _ae_body,
        grid=(rows // blk,),
        in_specs=[
            pl.BlockSpec((blk, 4), lambda i: (i, 0)),
            pl.BlockSpec((4, 16), lambda i: (0, 0)),
        ],
        out_specs=pl.BlockSpec((blk, 16), lambda i: (i, 0)),
        out_shape=jax.ShapeDtypeStruct((rows, 16), jnp.float32),
    )(ed4, cmat)


def _sc_edge_body(xh_hbm, asds_hbm, asdd_hbm, ae_hbm, src_hbm, dst_hbm,
                  out_hbm, srcv, dstv, srcg2, dst2v,
                  aev, asv, adv, xrow, xrow2, msgbuf, msgbuf2, z16, z64,
                  asds_sh, asdd_sh, den_sh, out_sh, sem, sem2):
    c = lax.axis_index("c")
    s = lax.axis_index("s")
    ept = srcv.shape[0]
    nch = ept // GCH
    nrows = N_NODES // NTILE
    base_e = s * ept

    pltpu.sync_copy(src_hbm.at[pl.ds(base_e, ept)], srcv)
    pltpu.sync_copy(dst_hbm.at[pl.ds(base_e, ept)], dstv)
    pltpu.sync_copy(ae_hbm.at[pl.ds(base_e, ept)], aev)

    def d2loop(k, carry):
        for j in range(GCH // 16):
            dst2v[k, pl.ds(j * 16, 16)] = dstv[pl.ds(k * GCH + j * 16, 16)]
        return carry

    lax.fori_loop(0, ept // GCH, d2loop, 0)

    def zloop16(i, carry):
        z16[i] = jnp.zeros((16,), jnp.float32)
        return carry

    lax.fori_loop(0, z16.shape[0], zloop16, 0)

    def zloop64(i, carry):
        for j in range(HID // 16):
            z64[i, pl.ds(j * 16, 16)] = jnp.zeros((16,), jnp.float32)
        return carry

    lax.fori_loop(0, z64.shape[0], zloop64, 0)

    def batch_body(b, carry):
        bg = c * (B // NSC) + b
        off = bg * N_NODES

        def idx2loop(k, cc):
            for j in range(GCH // 16):
                srcg2[k, pl.ds(j * 16, 16)] = (
                    srcv[pl.ds(k * GCH + j * 16, 16)] + off)
            return cc

        lax.fori_loop(0, nch, idx2loop, 0)

        cps = [
            pltpu.async_copy(z16, den_sh.at[pl.ds(s * nrows, nrows)], sem),
        ]
        for j in range(nrows // z64.shape[0]):
            cps.append(pltpu.async_copy(
                z64, out_sh.at[pl.ds(s * nrows + j * z64.shape[0],
                                     z64.shape[0])], sem))
        cps.append(pltpu.async_copy(
            asds_hbm.at[pl.ds(off + s * nrows, nrows)],
            asds_sh.at[pl.ds(s * nrows, nrows)], sem))
        cps.append(pltpu.async_copy(
            asdd_hbm.at[pl.ds(off + s * nrows, nrows)],
            asdd_sh.at[pl.ds(s * nrows, nrows)], sem))
        for cp in cps:
            cp.wait()
        plsc.subcore_barrier()

        cp_s = pltpu.async_copy(asds_sh.at[srcv], asv, sem)
        cp_d = pltpu.async_copy(asdd_sh.at[dstv], adv, sem2)
        cp_s.wait()
        cp_d.wait()

        def exloop(e8, cc):
            for u in range(8):
                e = e8 * 8 + u
                a = asv[e] + adv[e] + aev[e]
                a = jnp.where(a >= 0, a, 0.2 * a)
                asv[e] = jnp.exp(a)
            return cc

        lax.fori_loop(0, ept // 8, exloop, 0)

        pltpu.sync_copy(asv, den_sh.at[dstv], add=True)
        plsc.subcore_barrier()

        pltpu.async_copy(den_sh.at[dstv], adv, sem).wait()

        def wloop(e8, cc):
            for u in range(8):
                e = e8 * 8 + u
                asv[e] = asv[e] / (4.0 * adv[e] + 4e-16)
            return cc

        lax.fori_loop(0, ept // 8, wloop, 0)

        def compute_chunk(k, buf, mbuf):
            for i in range(GCH):
                wrow = asv[k * GCH + i]
                wh = [jnp.broadcast_to(lax.slice(wrow, (h,), (h + 1,)), (16,))
                      for h in range(HEADS)]
                for j in range(HID // 16):
                    acc = wh[0] * buf[i, pl.ds(j * 16, 16)]
                    for h in range(1, HEADS):
                        acc = acc + wh[h] * buf[i, pl.ds(h * HID + j * 16, 16)]
                    mbuf[i, pl.ds(j * 16, 16)] = acc
            pltpu.sync_copy(mbuf, out_sh.at[dst2v.at[k]], add=True)

        pltpu.async_copy(xh_hbm.at[srcg2.at[0]], xrow, sem)

        def pair_body(p, cc):
            ka = 2 * p
            kb = 2 * p + 1
            pltpu.make_async_copy(xh_hbm.at[srcg2.at[0]], xrow, sem).wait()
            pltpu.async_copy(xh_hbm.at[srcg2.at[kb]], xrow2, sem2)
            compute_chunk(ka, xrow, msgbuf)
            pltpu.make_async_copy(xh_hbm.at[srcg2.at[0]], xrow2, sem2).wait()
            knext = jnp.minimum(ka + 2, nch - 1)
            pltpu.async_copy(xh_hbm.at[srcg2.at[knext]], xrow, sem)
            compute_chunk(kb, xrow2, msgbuf2)
            return cc

        lax.fori_loop(0, nch // 2, pair_body, 0)
        pltpu.make_async_copy(xh_hbm.at[srcg2.at[0]], xrow, sem).wait()
        plsc.subcore_barrier()

        pltpu.sync_copy(out_sh.at[pl.ds(s * nrows, nrows)],
                        out_hbm.at[pl.ds(off + s * nrows, nrows)])
        return carry

    lax.fori_loop(0, B // NSC, batch_body, 0)


def _sc_edge(xh, asds, asdd, ae16, src_i, dst_i):
    bn = xh.shape[0]
    E = src_i.shape[0]
    ept = E // NTILE
    fn = pl.kernel(
        _sc_edge_body,
        out_type=pltpu.MemorySpace.HBM((bn, HID), jnp.float32),
        mesh=plsc.VectorSubcoreMesh(core_axis_name="c", subcore_axis_name="s"),
        compiler_params=pltpu.CompilerParams(use_tc_tiling_on_sc=False),
        scratch_types=[
            pltpu.VMEM((ept,), jnp.int32),
            pltpu.VMEM((ept,), jnp.int32),
            pltpu.VMEM((ept // GCH, GCH), jnp.int32),
            pltpu.VMEM((ept // GCH, GCH), jnp.int32),
            pltpu.VMEM((ept, 16), jnp.float32),
            pltpu.VMEM((ept, 16), jnp.float32),
            pltpu.VMEM((ept, 16), jnp.float32),
            pltpu.VMEM((GCH, HEADS * HID), jnp.float32),
            pltpu.VMEM((GCH, HEADS * HID), jnp.float32),
            pltpu.VMEM((GCH, HID), jnp.float32),
            pltpu.VMEM((GCH, HID), jnp.float32),
            pltpu.VMEM((N_NODES // NTILE, 16), jnp.float32),
            pltpu.VMEM((GCH, HID), jnp.float32),
            pltpu.VMEM_SHARED((N_NODES, 16), jnp.float32),
            pltpu.VMEM_SHARED((N_NODES, 16), jnp.float32),
            pltpu.VMEM_SHARED((N_NODES, 16), jnp.float32),
            pltpu.VMEM_SHARED((N_NODES, HID), jnp.float32),
            pltpu.SemaphoreType.DMA,
            pltpu.SemaphoreType.DMA,
        ],
    )
    return fn(xh, asds, asdd, ae16, src_i, dst_i)


def _finhead_body(h_ref, g1_ref, g2_ref, g3_ref, g4_ref, bias_ref, lng_ref,
                  lnb_ref, flag_ref, hlng_ref, hlnb_ref, w1_ref, b1_ref,
                  w2_ref, b2_ref, out_ref):
    def post(g):
        return _silu(_layernorm(g + bias_ref[...], lng_ref[...], lnb_ref[...]))

    k1 = post(g1_ref[...])
    k2 = post(g2_ref[...])
    k3 = post(g3_ref[...])
    k4 = post(g4_ref[...])
    h = h_ref[...] + (1.0 / 6.0) * (k1 + 2 * k2 + 2 * k3 + k4)
    z = _layernorm(h, hlng_ref[...], hlnb_ref[...])
    z = _silu(jnp.dot(z, w1_ref[...], preferred_element_type=jnp.float32)
              + b1_ref[...])
    logits = (z * w2_ref[...]).sum(-1, keepdims=True) + b2_ref[...]
    out_ref[...] = jnp.where(flag_ref[...] > 0.5,
                             jnp.maximum(logits, 6.0), logits)


def _finhead(h, g1, g2, g3, g4, bias, lng, lnb, flag, hlng, hlnb, w1, b1,
             w2, b2):
    rows = h.shape[0]
    blk = min(2048, rows)
    row_spec = pl.BlockSpec((blk, HID), lambda i: (i, 0))
    vec_spec = pl.BlockSpec((1, HID), lambda i: (0, 0))
    return pl.pallas_call(
        _finhead_body,
        grid=(rows // blk,),
        in_specs=[row_spec] * 5 + [vec_spec] * 3 + [
            pl.BlockSpec((blk, 1), lambda i: (i, 0)),
            vec_spec, vec_spec,
            pl.BlockSpec((HID, HID), lambda i: (0, 0)),
            vec_spec, vec_spec,
            pl.BlockSpec((1, 1), lambda i: (0, 0)),
        ],
        out_specs=pl.BlockSpec((blk, 1), lambda i: (i, 0)),
        out_shape=jax.ShapeDtypeStruct((rows, 1), jnp.float32),
    )(h, g1, g2, g3, g4, bias, lng, lnb, flag, hlng, hlnb, w1, b1, w2, b2)


def kernel(x, edge_index, edge_dirs, params):
    E = edge_index.shape[1]
    n_rows = B * N_NODES
    nodes = x.reshape(B, IN_DIM, N_NODES).transpose(0, 2, 1)
    nodes = nodes.reshape(n_rows, IN_DIM).astype(jnp.float32)

    src_i = edge_index[0].astype(jnp.int32)
    dst_i = edge_index[1].astype(jnp.int32)
    ed4 = jnp.concatenate(
        [edge_dirs, jnp.zeros((E, 1), jnp.float32)], axis=1)

    p = params
    gat_pack = []
    for gp in p['gats']:
        ws = jnp.zeros((HEADS * HID, 16), jnp.float32)
        wd = jnp.zeros((HEADS * HID, 16), jnp.float32)
        for h in range(HEADS):
            ws = ws.at[h * HID:(h + 1) * HID, h].set(gp['att_src'][h])
            wd = wd.at[h * HID:(h + 1) * HID, h].set(gp['att_dst'][h])
        lew = gp['lin_edge_W'].reshape(3, HEADS, HID)
        c34 = jnp.einsum('jhk,hk->jh', lew, gp['att_edge'])
        cmat = jnp.zeros((4, 16), jnp.float32).at[:3, :4].set(c34)
        gat_pack.append({
            'lin_W': gp['lin_W'], 'ws': ws, 'wd': wd, 'cmat': cmat,
            'bias': gp['bias'].reshape(1, HID),
            'ln_g': gp['ln_g'].reshape(1, HID),
            'ln_b': gp['ln_b'].reshape(1, HID),
        })

    ae_l = [_ae16(ed4, gat_pack[0]['cmat']), _ae16(ed4, gat_pack[1]['cmat'])]

    h0 = _encode(nodes, p['enc_W1'], p['enc_b1'].reshape(1, HID),
                 p['enc_W2'], p['enc_b2'].reshape(1, HID))

    coefs = [(1.0, 0.0), (1.0, 0.5), (1.0, 0.5), (1.0, 1.0)]
    g2_list = []
    g2_prev = None
    for i in range(4):
        ca, cb = coefs[i]
        gp0, gp1 = gat_pack
        xh, asds, asdd = _pre(h0, g2_prev, gp1['bias'], gp1['ln_g'],
                              gp1['ln_b'], gp0['lin_W'], gp0['ws'], gp0['wd'],
                              ca, cb, mode=("plain" if i == 0 else "mix"))
        g1 = _sc_edge(xh, asds, asdd, ae_l[0], src_i, dst_i)
        xh2, asds2, asdd2 = _pre(None, g1, gp0['bias'], gp0['ln_g'],
                                 gp0['ln_b'], gp1['lin_W'], gp1['ws'],
                                 gp1['wd'], 0.0, 1.0, mode="post")
        g2 = _sc_edge(xh2, asds2, asdd2, ae_l[1], src_i, dst_i)
        g2_list.append(g2)
        g2_prev = g2

    gp1 = gat_pack[1]
    flag = x[:, 0].reshape(n_rows, 1)
    logits = _finhead(h0, g2_list[0], g2_list[1], g2_list[2], g2_list[3],
                      gp1['bias'], gp1['ln_g'], gp1['ln_b'], flag,
                      p['head_ln_g'].reshape(1, HID),
                      p['head_ln_b'].reshape(1, HID), p['head_W1'],
                      p['head_b1'].reshape(1, HID),
                      p['head_W2'].reshape(1, HID),
                      p['head_b2'].reshape(1, 1))
    return logits.reshape(B, x.shape[2], x.shape[3])
